# baseline (speedup 1.0000x reference)
import functools
import jax
import jax.numpy as jnp
from jax import lax
from jax.experimental import pallas as pl
from jax.experimental.pallas import tpu as pltpu
from jax.experimental.pallas import tpu_sc as plsc

BATCH = 2048
IN_DIM = 4096
OUT_DIM = 8192
L = 16
NW = 32
ROWS_PER_W = BATCH // NW
R = 4
NBLK = ROWS_PER_W // R
NGC = OUT_DIM // L


def _sc_body(x_hbm, la_hbm, lb_hbm, ia_hbm, ib_hbm, out_hbm,
             ia_v, ib_v, ca_v, sa_v, cb_v, sb_v, xblk_v, oblk_v):
    wid = lax.axis_index("s") * 2 + lax.axis_index("c")
    row_base = wid * ROWS_PER_W

    pltpu.sync_copy(ia_hbm, ia_v)
    pltpu.sync_copy(ib_hbm, ib_v)
    pltpu.sync_copy(la_hbm, ca_v)
    pltpu.sync_copy(lb_hbm, cb_v)

    def init_consts(gc, _):
        s = pl.ds(gc * L, L)
        la = ca_v[s]
        lb = cb_v[s]
        ca = jnp.where(la > 0.0, 1.0, 0.0)
        cb = jnp.where(lb > 0.0, 1.0, 0.0)
        ca_v[s] = ca
        sa_v[s] = 1.0 - 2.0 * ca
        cb_v[s] = cb
        sb_v[s] = 1.0 - 2.0 * cb
        return 0

    lax.fori_loop(0, NGC, init_consts, 0)

    def do_block(blk, _):
        row0 = row_base + blk * R
        pltpu.sync_copy(x_hbm.at[pl.ds(row0 * IN_DIM, R * IN_DIM)], xblk_v)

        def do_chunk(gc, _):
            s = pl.ds(gc * L, L)
            ia = ia_v[s]
            ib = ib_v[s]
            ca = ca_v[s]
            sa = sa_v[s]
            cb = cb_v[s]
            sb = sb_v[s]
            for r in range(R):
                a = plsc.load_gather(xblk_v, [ia + (r * IN_DIM)])
                b = plsc.load_gather(xblk_v, [ib + (r * IN_DIM)])
                oblk_v[pl.ds(r * OUT_DIM + gc * L, L)] = (
                    (ca + sa * a) * (cb + sb * b))
            return 0

        lax.fori_loop(0, NGC, do_chunk, 0)
        pltpu.sync_copy(oblk_v, out_hbm.at[pl.ds(row0 * OUT_DIM, R * OUT_DIM)])
        return 0

    lax.fori_loop(0, NBLK, do_block, 0)


@jax.jit
def kernel(x, negation_logits, idx_a, idx_b):
    la = negation_logits[:, 0].copy()
    lb = negation_logits[:, 1].copy()
    mesh = plsc.VectorSubcoreMesh(core_axis_name="c", subcore_axis_name="s")
    f = pl.kernel(
        _sc_body,
        out_type=jax.ShapeDtypeStruct((BATCH * OUT_DIM,), jnp.float32),
        mesh=mesh,
        compiler_params=pltpu.CompilerParams(needs_layout_passes=False),
        scratch_types=[
            pltpu.VMEM((OUT_DIM,), jnp.int32),
            pltpu.VMEM((OUT_DIM,), jnp.int32),
            pltpu.VMEM((OUT_DIM,), jnp.float32),
            pltpu.VMEM((OUT_DIM,), jnp.float32),
            pltpu.VMEM((OUT_DIM,), jnp.float32),
            pltpu.VMEM((OUT_DIM,), jnp.float32),
            pltpu.VMEM((R * IN_DIM,), jnp.float32),
            pltpu.VMEM((R * OUT_DIM,), jnp.float32),
        ],
    )
    out = f(x.reshape(-1), la, lb, idx_a, idx_b)
    return out.reshape(BATCH, OUT_DIM)


# speedup vs baseline: 1.0357x; 1.0357x over previous
import functools
import jax
import jax.numpy as jnp
from jax import lax
from jax.experimental import pallas as pl
from jax.experimental.pallas import tpu as pltpu
from jax.experimental.pallas import tpu_sc as plsc

BATCH = 2048
IN_DIM = 4096
OUT_DIM = 8192
L = 16
NW = 32
ROWS_PER_W = BATCH // NW
R = 4
NBLK = ROWS_PER_W // R
NGC = OUT_DIM // L
SIGN = jnp.int32(-2147483648)
MASK = jnp.int32(0x7FFFFFFF)


def _sc_body(x_hbm, ll_hbm, ia_hbm, ib_hbm, out_hbm,
             pia_v, pib_v, xblk_v, oblk_v,
             xsem0, xsem1, osem0, osem1):
    wid = lax.axis_index("s") * 2 + lax.axis_index("c")
    row_base = wid * ROWS_PER_W
    xsems = (xsem0, xsem1)
    osems = (osem0, osem1)

    pltpu.sync_copy(ia_hbm, pia_v)
    pltpu.sync_copy(ib_hbm, pib_v)
    pltpu.sync_copy(ll_hbm, oblk_v.at[pl.ds(0, 2 * OUT_DIM)])

    @pl.loop(0, NGC)
    def init_consts(gc):
        s = pl.ds(gc * L, L)
        j2 = (gc * (2 * L)) + 2 * lax.iota(jnp.int32, 16)
        la = plsc.load_gather(oblk_v, [j2])
        lb = plsc.load_gather(oblk_v, [j2 + 1])
        pia_v[s] = pia_v[s] | jnp.where(la > 0.0, SIGN, 0)
        pib_v[s] = pib_v[s] | jnp.where(lb > 0.0, SIGN, 0)

    def x_copy(blk, b):
        off = (row_base + blk * R) * IN_DIM
        return pltpu.make_async_copy(
            x_hbm.at[pl.ds(off, R * IN_DIM)],
            xblk_v.at[pl.ds(b * R * IN_DIM, R * IN_DIM)],
            xsems[b])

    def o_copy(blk, b):
        off = (row_base + blk * R) * OUT_DIM
        return pltpu.make_async_copy(
            oblk_v.at[pl.ds(b * R * OUT_DIM, R * OUT_DIM)],
            out_hbm.at[pl.ds(off, R * OUT_DIM)],
            osems[b])

    x_copy(0, 0).start()
    x_copy(1, 1).start()

    @pl.loop(0, NBLK, step=2)
    def outer(blk0):
        for b in range(2):
            blk = blk0 + b
            x_copy(blk, b).wait()

            @pl.when(blk0 > 0)
            def _():
                o_copy(blk, b).wait()

            base_x = b * R * IN_DIM
            base_o = b * R * OUT_DIM

            @pl.loop(0, NGC, unroll=2)
            def do_chunk(gc):
                s = pl.ds(gc * L, L)
                pia = pia_v[s]
                pib = pib_v[s]
                ca = lax.shift_right_logical(pia, 31).astype(jnp.float32)
                cb = lax.shift_right_logical(pib, 31).astype(jnp.float32)
                sa = 1.0 - 2.0 * ca
                sb = 1.0 - 2.0 * cb
                ia = pia & MASK
                ib = pib & MASK
                for r in range(R):
                    a = plsc.load_gather(xblk_v, [ia + (base_x + r * IN_DIM)])
                    bb = plsc.load_gather(xblk_v, [ib + (base_x + r * IN_DIM)])
                    oblk_v[pl.ds(base_o + r * OUT_DIM + gc * L, L)] = (
                        (ca + sa * a) * (cb + sb * bb))

            o_copy(blk, b).start()

            @pl.when(blk + 2 < NBLK)
            def _():
                x_copy(blk + 2, b).start()

    o_copy(NBLK - 2, 0).wait()
    o_copy(NBLK - 1, 1).wait()


@jax.jit
def kernel(x, negation_logits, idx_a, idx_b):
    mesh = plsc.VectorSubcoreMesh(core_axis_name="c", subcore_axis_name="s")
    f = pl.kernel(
        _sc_body,
        out_type=jax.ShapeDtypeStruct((BATCH * OUT_DIM,), jnp.float32),
        mesh=mesh,
        compiler_params=pltpu.CompilerParams(needs_layout_passes=False),
        scratch_types=[
            pltpu.VMEM((OUT_DIM,), jnp.int32),
            pltpu.VMEM((OUT_DIM,), jnp.int32),
            pltpu.VMEM((2 * R * IN_DIM,), jnp.float32),
            pltpu.VMEM((2 * R * OUT_DIM,), jnp.float32),
            pltpu.SemaphoreType.DMA,
            pltpu.SemaphoreType.DMA,
            pltpu.SemaphoreType.DMA,
            pltpu.SemaphoreType.DMA,
        ],
    )
    out = f(x.reshape(-1), negation_logits.reshape(-1), idx_a, idx_b)
    return out.reshape(BATCH, OUT_DIM)


# speedup vs baseline: 3.0033x; 2.8997x over previous
import functools
import jax
import jax.numpy as jnp
from jax import lax
from jax.experimental import pallas as pl
from jax.experimental.pallas import tpu as pltpu
from jax.experimental.pallas import tpu_sc as plsc

BATCH = 2048
IN_DIM = 4096
OUT_DIM = 8192
L = 16
NW = 32
ROWS_PER_W = BATCH // NW
R = 4
NBLK = ROWS_PER_W // R
NGC = OUT_DIM // L
SIGN = jnp.int32(-2147483648)
MASK = jnp.int32(0x7FFFFFFF)


def _sc_body(x_hbm, ll_hbm, ia_hbm, ib_hbm, out_hbm,
             pia_v, pib_v, xblk_v, oblk_v,
             xsem0, xsem1, osem0, osem1):
    wid = lax.axis_index("s") * 2 + lax.axis_index("c")
    row_base = wid * ROWS_PER_W
    xsems = (xsem0, xsem1)
    osems = (osem0, osem1)

    pltpu.sync_copy(ia_hbm, pia_v)
    pltpu.sync_copy(ib_hbm, pib_v)
    pltpu.sync_copy(ll_hbm, oblk_v.at[pl.ds(0, 2 * OUT_DIM)])

    @pl.loop(0, NGC)
    def init_consts(gc):
        s = pl.ds(gc * L, L)
        j2 = (gc * (2 * L)) + 2 * lax.iota(jnp.int32, 16)
        la = plsc.load_gather(oblk_v, [j2])
        lb = plsc.load_gather(oblk_v, [j2 + 1])
        pia_v[s] = pia_v[s] | jnp.where(la > 0.0, SIGN, 0)
        pib_v[s] = pib_v[s] | jnp.where(lb > 0.0, SIGN, 0)

    def x_copy(blk, b):
        off = (row_base + blk * R) * IN_DIM
        return pltpu.make_async_copy(
            x_hbm.at[pl.ds(off, R * IN_DIM)],
            xblk_v.at[pl.ds(b * R * IN_DIM, R * IN_DIM)],
            xsems[b])

    def o_copy(blk, b):
        off = (row_base + blk * R) * OUT_DIM
        return pltpu.make_async_copy(
            oblk_v.at[pl.ds(b * R * OUT_DIM, R * OUT_DIM)],
            out_hbm.at[pl.ds(off, R * OUT_DIM)],
            osems[b])

    x_copy(0, 0).start()
    x_copy(1, 1).start()

    @pl.loop(0, NBLK, step=2)
    def outer(blk0):
        for b in range(2):
            blk = blk0 + b
            x_copy(blk, b).wait()

            @pl.when(blk0 > 0)
            def _():
                o_copy(blk, b).wait()

            base_x = b * R * IN_DIM
            base_o = b * R * OUT_DIM

            @functools.partial(plsc.parallel_loop, 0, NGC, unroll=4)
            def do_chunk(gc):
                s = pl.ds(gc * L, L)
                pia = pia_v[s]
                pib = pib_v[s]
                ca = lax.shift_right_logical(pia, 31).astype(jnp.float32)
                cb = lax.shift_right_logical(pib, 31).astype(jnp.float32)
                sa = 1.0 - 2.0 * ca
                sb = 1.0 - 2.0 * cb
                ia = pia & MASK
                ib = pib & MASK
                for r in range(R):
                    a = plsc.load_gather(xblk_v, [ia + (base_x + r * IN_DIM)])
                    bb = plsc.load_gather(xblk_v, [ib + (base_x + r * IN_DIM)])
                    oblk_v[pl.ds(base_o + r * OUT_DIM + gc * L, L)] = (
                        (ca + sa * a) * (cb + sb * bb))

            o_copy(blk, b).start()

            @pl.when(blk + 2 < NBLK)
            def _():
                x_copy(blk + 2, b).start()

    o_copy(NBLK - 2, 0).wait()
    o_copy(NBLK - 1, 1).wait()


@jax.jit
def kernel(x, negation_logits, idx_a, idx_b):
    mesh = plsc.VectorSubcoreMesh(core_axis_name="c", subcore_axis_name="s")
    f = pl.kernel(
        _sc_body,
        out_type=jax.ShapeDtypeStruct((BATCH * OUT_DIM,), jnp.float32),
        mesh=mesh,
        compiler_params=pltpu.CompilerParams(needs_layout_passes=False),
        scratch_types=[
            pltpu.VMEM((OUT_DIM,), jnp.int32),
            pltpu.VMEM((OUT_DIM,), jnp.int32),
            pltpu.VMEM((2 * R * IN_DIM,), jnp.float32),
            pltpu.VMEM((2 * R * OUT_DIM,), jnp.float32),
            pltpu.SemaphoreType.DMA,
            pltpu.SemaphoreType.DMA,
            pltpu.SemaphoreType.DMA,
            pltpu.SemaphoreType.DMA,
        ],
    )
    out = f(x.reshape(-1), negation_logits.reshape(-1), idx_a, idx_b)
    return out.reshape(BATCH, OUT_DIM)
